# baseline (speedup 1.0000x reference)
import functools

import jax
import jax.numpy as jnp
from jax import lax
from jax.experimental import pallas as pl
from jax.experimental.pallas import tpu as pltpu
from jax.experimental.pallas import tpu_sc as plsc


@functools.lru_cache(maxsize=None)
def _make_sc_gather(V, D, B):
    info = plsc.get_sparse_core_info()
    NC, NS = info.num_cores, info.num_subcores
    NW = NC * NS
    b_per_w = B // NW
    mesh = plsc.VectorSubcoreMesh(core_axis_name="c", subcore_axis_name="s")

    @functools.partial(
        pl.kernel,
        mesh=mesh,
        out_type=jax.ShapeDtypeStruct((B, D), jnp.float32),
        scratch_types=[
            pltpu.VMEM((b_per_w,), jnp.int32),
            pltpu.VMEM((b_per_w, D), jnp.float32),
            pltpu.SemaphoreType.DMA,
        ],
        compiler_params=pltpu.CompilerParams(use_tc_tiling_on_sc=False),
    )
    def gather_kernel(table_hbm, idx_hbm, out_hbm, idx_v, rows_v, sem):
        wid = lax.axis_index("s") * NC + lax.axis_index("c")
        base = wid * b_per_w
        pltpu.sync_copy(idx_hbm.at[pl.ds(base, b_per_w)], idx_v)
        pltpu.async_copy(table_hbm.at[idx_v], rows_v, sem).wait()
        pltpu.sync_copy(rows_v, out_hbm.at[pl.ds(base, b_per_w)])

    return gather_kernel


_BV = 2048


def _mlp_body(h_raw_ref, w1_ref, b1_ref, w2_ref, b2_ref, out_ref, h_scr):
    @pl.when(pl.program_id(0) == 0)
    def _():
        h = jnp.dot(h_raw_ref[...], w1_ref[...],
                    preferred_element_type=jnp.float32) + b1_ref[...]
        h_scr[...] = jnp.maximum(h, 0.0)

    out_ref[...] = (
        jnp.dot(h_scr[...], w2_ref[...], preferred_element_type=jnp.float32)
        + b2_ref[...]
    )


def kernel(x, emb, W1, b1, W2, b2):
    V, H = emb.shape
    B = x.shape[0]
    idx = x.astype(jnp.int32)
    h_raw = _make_sc_gather(V, H, B)(emb, idx)

    grid = pl.cdiv(V, _BV)
    out = pl.pallas_call(
        _mlp_body,
        grid=(grid,),
        in_specs=[
            pl.BlockSpec((B, H), lambda i: (0, 0)),
            pl.BlockSpec((H, H), lambda i: (0, 0)),
            pl.BlockSpec((1, H), lambda i: (0, 0)),
            pl.BlockSpec((H, _BV), lambda i: (0, i)),
            pl.BlockSpec((1, _BV), lambda i: (0, i)),
        ],
        out_specs=pl.BlockSpec((B, _BV), lambda i: (0, i)),
        out_shape=jax.ShapeDtypeStruct((B, V), jnp.float32),
        scratch_shapes=[pltpu.VMEM((B, H), jnp.float32)],
    )(h_raw, W1, b1.reshape(1, H), W2, b2.reshape(1, V))
    return out


# speedup vs baseline: 2.0497x; 2.0497x over previous
import functools

import jax
import jax.numpy as jnp
from jax import lax
from jax.experimental import pallas as pl
from jax.experimental.pallas import tpu as pltpu
from jax.experimental.pallas import tpu_sc as plsc


@functools.lru_cache(maxsize=None)
def _make_sc_gather(V, D, B):
    info = plsc.get_sparse_core_info()
    NC, NS = info.num_cores, info.num_subcores
    NW = NC * NS
    b_per_w = B // NW
    mesh = plsc.VectorSubcoreMesh(core_axis_name="c", subcore_axis_name="s")

    @functools.partial(
        pl.kernel,
        mesh=mesh,
        out_type=jax.ShapeDtypeStruct((B, D), jnp.float32),
        scratch_types=[
            pltpu.VMEM((b_per_w,), jnp.int32),
            pltpu.VMEM((b_per_w, D), jnp.float32),
            pltpu.SemaphoreType.DMA,
        ],
        compiler_params=pltpu.CompilerParams(use_tc_tiling_on_sc=False),
    )
    def gather_kernel(table_hbm, idx_hbm, out_hbm, idx_v, rows_v, sem):
        wid = lax.axis_index("s") * NC + lax.axis_index("c")
        base = wid * b_per_w
        pltpu.sync_copy(idx_hbm.at[pl.ds(base, b_per_w)], idx_v)
        pltpu.async_copy(table_hbm.at[idx_v], rows_v, sem).wait()
        pltpu.sync_copy(rows_v, out_hbm.at[pl.ds(base, b_per_w)])

    return gather_kernel


_BV = 2048


def _mlp_body(h_raw_ref, w1_ref, b1_ref, w2_ref, b2_ref, outT_ref, ht_scr):
    @pl.when(pl.program_id(0) == 0)
    def _():
        h = jnp.dot(h_raw_ref[...], w1_ref[...],
                    preferred_element_type=jnp.float32) + b1_ref[...]
        ht_scr[...] = jnp.maximum(h, 0.0).T

    outT_ref[...] = (
        lax.dot_general(w2_ref[...], ht_scr[...], (((0,), (0,)), ((), ())),
                        preferred_element_type=jnp.float32)
        + b2_ref[...]
    )


def kernel(x, emb, W1, b1, W2, b2):
    V, H = emb.shape
    B = x.shape[0]
    idx = x.astype(jnp.int32)
    h_raw = _make_sc_gather(V, H, B)(emb, idx)

    grid = pl.cdiv(V, _BV)
    outT = pl.pallas_call(
        _mlp_body,
        grid=(grid,),
        in_specs=[
            pl.BlockSpec((B, H), lambda i: (0, 0)),
            pl.BlockSpec((H, H), lambda i: (0, 0)),
            pl.BlockSpec((1, H), lambda i: (0, 0)),
            pl.BlockSpec((H, _BV), lambda i: (0, i)),
            pl.BlockSpec((_BV, 1), lambda i: (i, 0)),
        ],
        out_specs=pl.BlockSpec((_BV, B), lambda i: (i, 0)),
        out_shape=jax.ShapeDtypeStruct((V, B), jnp.float32),
        scratch_shapes=[pltpu.VMEM((H, B), jnp.float32)],
    )(h_raw, W1, b1.reshape(1, H), W2, b2.reshape(V, 1))
    return outT.T


# speedup vs baseline: 2.5218x; 1.2303x over previous
import functools

import jax
import jax.numpy as jnp
from jax import lax
from jax.experimental import pallas as pl
from jax.experimental.pallas import tpu as pltpu
from jax.experimental.pallas import tpu_sc as plsc


_RB = 512


def _prepass_body(emb8_ref, w1_ref, b1c_ref, out_ref):
    ys = [
        jnp.dot(emb8_ref[:, k, :], w1_ref[...],
                preferred_element_type=jnp.float32)
        for k in range(8)
    ]
    out_ref[...] = jnp.concatenate(ys, axis=1) + b1c_ref[...]


@functools.lru_cache(maxsize=None)
def _make_sc_gather(VG, DG, B):
    D = DG // 8
    info = plsc.get_sparse_core_info()
    NC, NS, L = info.num_cores, info.num_subcores, info.num_lanes
    NW = 16
    b_per_w = B // NW
    n_grp = b_per_w // L
    mesh = plsc.VectorSubcoreMesh(core_axis_name="c", subcore_axis_name="s")

    @functools.partial(
        pl.kernel,
        mesh=mesh,
        out_type=jax.ShapeDtypeStruct((B // 8, DG), jnp.float32),
        scratch_types=[
            pltpu.VMEM((b_per_w,), jnp.int32),
            pltpu.VMEM((b_per_w,), jnp.int32),
            pltpu.VMEM((b_per_w, DG), jnp.float32),
            pltpu.VMEM((b_per_w // 8, DG), jnp.float32),
            pltpu.SemaphoreType.DMA,
        ],
        compiler_params=pltpu.CompilerParams(needs_layout_passes=False),
    )
    def gather_kernel(table_hbm, idx_hbm, out_hbm, idx_v, maj_v, tiles_v,
                      rows_v, sem):
        wid = lax.axis_index("s") * NC + lax.axis_index("c")

        @pl.when(wid < NW)
        def _():
            base = wid * b_per_w
            pltpu.sync_copy(idx_hbm.at[pl.ds(base, b_per_w)], idx_v)
            for g in range(n_grp):
                maj_v[pl.ds(g * L, L)] = jax.lax.shift_right_logical(
                    idx_v[pl.ds(g * L, L)], 3)
            pltpu.async_copy(table_hbm.at[maj_v], tiles_v, sem).wait()
            for g in range(n_grp):
                lane_t = lax.iota(jnp.int32, L) + g * L
                seg = jax.lax.rem(idx_v[pl.ds(g * L, L)], jnp.int32(8)) * D
                dst_r = jax.lax.shift_right_logical(lane_t, 3)
                dst_seg = jax.lax.rem(lane_t, jnp.int32(8)) * D

                def body(c, carry):
                    vals = plsc.load_gather(tiles_v, [lane_t, seg + c])
                    plsc.store_scatter(rows_v, [dst_r, dst_seg + c], vals)
                    return carry

                lax.fori_loop(0, D, body, jnp.int32(0))
            pltpu.sync_copy(
                rows_v, out_hbm.at[pl.ds(wid * (b_per_w // 8), b_per_w // 8)])

    return gather_kernel


_BV = 2048


def _mlp_body(hm_ref, w2_ref, b2_ref, outT_ref, hp_scr, ht_scr):
    @pl.when(pl.program_id(0) == 0)
    def _():
        for k in range(8):
            hp_scr[pl.Slice(k, hm_ref.shape[0], 8), :] = (
                hm_ref[:, pl.ds(k * 64, 64)])
        ht_scr[...] = jnp.maximum(hp_scr[...], 0.0).T

    outT_ref[...] = (
        lax.dot_general(w2_ref[...], ht_scr[...], (((0,), (0,)), ((), ())),
                        preferred_element_type=jnp.float32)
        + b2_ref[...].T
    )


def kernel(x, emb, W1, b1, W2, b2):
    V, H = emb.shape
    B = x.shape[0]
    idx = x.astype(jnp.int32)

    emb8 = emb.reshape(V // 8, 8, H)
    b1cat = jnp.tile(b1, 8).reshape(1, 8 * H)
    table = pl.pallas_call(
        _prepass_body,
        grid=(pl.cdiv(V // 8, _RB),),
        in_specs=[
            pl.BlockSpec((_RB, 8, H), lambda i: (i, 0, 0)),
            pl.BlockSpec((H, H), lambda i: (0, 0)),
            pl.BlockSpec((1, 8 * H), lambda i: (0, 0)),
        ],
        out_specs=pl.BlockSpec((_RB, 8 * H), lambda i: (i, 0)),
        out_shape=jax.ShapeDtypeStruct((V // 8, 8 * H), jnp.float32),
    )(emb8, W1, b1cat)

    hm = _make_sc_gather(V // 8, 8 * H, B)(table, idx)

    grid = pl.cdiv(V, _BV)
    outT = pl.pallas_call(
        _mlp_body,
        grid=(grid,),
        in_specs=[
            pl.BlockSpec((B // 8, 8 * H), lambda i: (0, 0)),
            pl.BlockSpec((H, _BV), lambda i: (0, i)),
            pl.BlockSpec((1, _BV), lambda i: (0, i)),
        ],
        out_specs=pl.BlockSpec((_BV, B), lambda i: (i, 0)),
        out_shape=jax.ShapeDtypeStruct((V, B), jnp.float32),
        scratch_shapes=[pltpu.VMEM((B, H), jnp.float32),
                        pltpu.VMEM((H, B), jnp.float32)],
    )(hm, W2, b2.reshape(1, V))
    return outT.T


# speedup vs baseline: 2.8929x; 1.1472x over previous
import functools

import jax
import jax.numpy as jnp
from jax import lax
from jax.experimental import pallas as pl
from jax.experimental.pallas import tpu as pltpu
from jax.experimental.pallas import tpu_sc as plsc


_RB = 512


def _prepass_body(embt_ref, w1_ref, b1c_ref, out_ref, y_scr):
    y_scr[...] = lax.dot_general(embt_ref[...], w1_ref[...],
                                 (((0,), (0,)), ((), ())),
                                 preferred_element_type=jnp.float32)
    nb = out_ref.shape[0]
    out_ref[...] = jnp.concatenate(
        [y_scr[pl.Slice(k, nb, 8), :] for k in range(8)], axis=1
    ) + b1c_ref[...]


@functools.lru_cache(maxsize=None)
def _make_sc_gather(VG, DG, B):
    D = DG // 8
    info = plsc.get_sparse_core_info()
    NC, NS, L = info.num_cores, info.num_subcores, info.num_lanes
    NW = 16
    b_per_w = B // NW
    n_grp = b_per_w // L
    mesh = plsc.VectorSubcoreMesh(core_axis_name="c", subcore_axis_name="s")

    @functools.partial(
        pl.kernel,
        mesh=mesh,
        out_type=jax.ShapeDtypeStruct((B // 8, DG), jnp.float32),
        scratch_types=[
            pltpu.VMEM((b_per_w,), jnp.int32),
            pltpu.VMEM((b_per_w,), jnp.int32),
            pltpu.VMEM((b_per_w, DG), jnp.float32),
            pltpu.VMEM((b_per_w // 8, DG), jnp.float32),
            pltpu.SemaphoreType.DMA,
        ],
        compiler_params=pltpu.CompilerParams(needs_layout_passes=False),
    )
    def gather_kernel(table_hbm, idx_hbm, out_hbm, idx_v, maj_v, tiles_v,
                      rows_v, sem):
        wid = lax.axis_index("s") * NC + lax.axis_index("c")

        @pl.when(wid < NW)
        def _():
            base = wid * b_per_w
            pltpu.sync_copy(idx_hbm.at[pl.ds(base, b_per_w)], idx_v)
            for g in range(n_grp):
                maj_v[pl.ds(g * L, L)] = jax.lax.shift_right_logical(
                    idx_v[pl.ds(g * L, L)], 3)
            pltpu.async_copy(table_hbm.at[maj_v], tiles_v, sem).wait()
            for g in range(n_grp):
                lane_t = lax.iota(jnp.int32, L) + g * L
                seg = jax.lax.rem(idx_v[pl.ds(g * L, L)], jnp.int32(8)) * D
                dst_r = jax.lax.shift_right_logical(lane_t, 3)
                dst_seg = jax.lax.rem(lane_t, jnp.int32(8)) * D

                def body(c, carry):
                    vals = plsc.load_gather(tiles_v, [lane_t, seg + c])
                    plsc.store_scatter(rows_v, [dst_r, dst_seg + c], vals)
                    return carry

                lax.fori_loop(0, D, body, jnp.int32(0))
            pltpu.sync_copy(
                rows_v, out_hbm.at[pl.ds(wid * (b_per_w // 8), b_per_w // 8)])

    return gather_kernel


_BV = 2048


def _mlp_body(hm_ref, w2_ref, b2_ref, outT_ref, hp_scr, ht_scr):
    @pl.when(pl.program_id(0) == 0)
    def _():
        for k in range(8):
            hp_scr[pl.Slice(k, hm_ref.shape[0], 8), :] = (
                hm_ref[:, pl.ds(k * 64, 64)])
        ht_scr[...] = jnp.maximum(hp_scr[...], 0.0).T

    outT_ref[...] = (
        lax.dot_general(w2_ref[...], ht_scr[...], (((0,), (0,)), ((), ())),
                        preferred_element_type=jnp.float32)
        + b2_ref[...].T
    )


def kernel(x, emb, W1, b1, W2, b2):
    V, H = emb.shape
    B = x.shape[0]
    idx = x.astype(jnp.int32)

    embt = emb.T
    b1cat = jnp.tile(b1, 8).reshape(1, 8 * H)
    table = pl.pallas_call(
        _prepass_body,
        grid=(pl.cdiv(V // 8, _RB),),
        in_specs=[
            pl.BlockSpec((H, 8 * _RB), lambda i: (0, i)),
            pl.BlockSpec((H, H), lambda i: (0, 0)),
            pl.BlockSpec((1, 8 * H), lambda i: (0, 0)),
        ],
        out_specs=pl.BlockSpec((_RB, 8 * H), lambda i: (i, 0)),
        out_shape=jax.ShapeDtypeStruct((V // 8, 8 * H), jnp.float32),
        scratch_shapes=[pltpu.VMEM((8 * _RB, H), jnp.float32)],
    )(embt, W1, b1cat)

    hm = _make_sc_gather(V // 8, 8 * H, B)(table, idx)

    grid = pl.cdiv(V, _BV)
    outT = pl.pallas_call(
        _mlp_body,
        grid=(grid,),
        in_specs=[
            pl.BlockSpec((B // 8, 8 * H), lambda i: (0, 0)),
            pl.BlockSpec((H, _BV), lambda i: (0, i)),
            pl.BlockSpec((1, _BV), lambda i: (0, i)),
        ],
        out_specs=pl.BlockSpec((_BV, B), lambda i: (i, 0)),
        out_shape=jax.ShapeDtypeStruct((V, B), jnp.float32),
        scratch_shapes=[pltpu.VMEM((B, H), jnp.float32),
                        pltpu.VMEM((H, B), jnp.float32)],
    )(hm, W2, b2.reshape(1, V))
    return outT.T


# speedup vs baseline: 3.0342x; 1.0488x over previous
import functools

import jax
import jax.numpy as jnp
from jax import lax
from jax.experimental import pallas as pl
from jax.experimental.pallas import tpu as pltpu
from jax.experimental.pallas import tpu_sc as plsc


_RB = 1024


def _prepass_body(embt_ref, w1_ref, b1c_ref, out_ref, y_scr):
    y_scr[...] = lax.dot_general(embt_ref[...], w1_ref[...],
                                 (((0,), (0,)), ((), ())),
                                 preferred_element_type=jnp.float32)
    nb = out_ref.shape[0]
    out_ref[...] = jnp.concatenate(
        [y_scr[pl.Slice(k, nb, 8), :] for k in range(8)], axis=1
    ) + b1c_ref[...]


@functools.lru_cache(maxsize=None)
def _make_sc_gather(VG, DG, B):
    D = DG // 8
    info = plsc.get_sparse_core_info()
    NC, NS, L = info.num_cores, info.num_subcores, info.num_lanes
    NW = 16
    b_per_w = B // NW
    n_grp = b_per_w // L
    mesh = plsc.VectorSubcoreMesh(core_axis_name="c", subcore_axis_name="s")

    @functools.partial(
        pl.kernel,
        mesh=mesh,
        out_type=jax.ShapeDtypeStruct((B // 8, DG), jnp.float32),
        scratch_types=[
            pltpu.VMEM((b_per_w,), jnp.int32),
            pltpu.VMEM((b_per_w,), jnp.int32),
            pltpu.VMEM((b_per_w, DG), jnp.float32),
            pltpu.VMEM((b_per_w // 8, DG), jnp.float32),
            pltpu.SemaphoreType.DMA,
        ],
        compiler_params=pltpu.CompilerParams(needs_layout_passes=False),
    )
    def gather_kernel(table_hbm, idx_hbm, out_hbm, idx_v, maj_v, tiles_v,
                      rows_v, sem):
        wid = lax.axis_index("s") * NC + lax.axis_index("c")

        @pl.when(wid < NW)
        def _():
            base = wid * b_per_w
            pltpu.sync_copy(idx_hbm.at[pl.ds(base, b_per_w)], idx_v)
            for g in range(n_grp):
                maj_v[pl.ds(g * L, L)] = jax.lax.shift_right_logical(
                    idx_v[pl.ds(g * L, L)], 3)
            pltpu.async_copy(table_hbm.at[maj_v], tiles_v, sem).wait()
            for g in range(n_grp):
                lane_t = lax.iota(jnp.int32, L) + g * L
                seg = jax.lax.rem(idx_v[pl.ds(g * L, L)], jnp.int32(8)) * D
                dst_r = jax.lax.shift_right_logical(lane_t, 3)
                dst_seg = jax.lax.rem(lane_t, jnp.int32(8)) * D

                def body(c, carry):
                    vals = plsc.load_gather(tiles_v, [lane_t, seg + c])
                    plsc.store_scatter(rows_v, [dst_r, dst_seg + c], vals)
                    return carry

                lax.fori_loop(0, D, body, jnp.int32(0))
            pltpu.sync_copy(
                rows_v, out_hbm.at[pl.ds(wid * (b_per_w // 8), b_per_w // 8)])

    return gather_kernel


_BV = 4096


def _mlp_body(hm_ref, w2_ref, b2_ref, outT_ref, hp_scr, ht_scr):
    @pl.when(pl.program_id(0) == 0)
    def _():
        for k in range(8):
            hp_scr[pl.Slice(k, hm_ref.shape[0], 8), :] = (
                hm_ref[:, pl.ds(k * 64, 64)])
        ht_scr[...] = jnp.maximum(hp_scr[...], 0.0).T

    outT_ref[...] = (
        lax.dot_general(w2_ref[...], ht_scr[...], (((0,), (0,)), ((), ())),
                        preferred_element_type=jnp.float32)
        + b2_ref[...].T
    )


def kernel(x, emb, W1, b1, W2, b2):
    V, H = emb.shape
    B = x.shape[0]
    idx = x.astype(jnp.int32)

    embt = emb.T
    b1cat = jnp.tile(b1, 8).reshape(1, 8 * H)
    table = pl.pallas_call(
        _prepass_body,
        grid=(pl.cdiv(V // 8, _RB),),
        in_specs=[
            pl.BlockSpec((H, 8 * _RB), lambda i: (0, i)),
            pl.BlockSpec((H, H), lambda i: (0, 0)),
            pl.BlockSpec((1, 8 * H), lambda i: (0, 0)),
        ],
        out_specs=pl.BlockSpec((_RB, 8 * H), lambda i: (i, 0)),
        out_shape=jax.ShapeDtypeStruct((V // 8, 8 * H), jnp.float32),
        scratch_shapes=[pltpu.VMEM((8 * _RB, H), jnp.float32)],
    )(embt, W1, b1cat)

    hm = _make_sc_gather(V // 8, 8 * H, B)(table, idx)

    grid = pl.cdiv(V, _BV)
    outT = pl.pallas_call(
        _mlp_body,
        grid=(grid,),
        in_specs=[
            pl.BlockSpec((B // 8, 8 * H), lambda i: (0, 0)),
            pl.BlockSpec((H, _BV), lambda i: (0, i)),
            pl.BlockSpec((1, _BV), lambda i: (0, i)),
        ],
        out_specs=pl.BlockSpec((_BV, B), lambda i: (i, 0)),
        out_shape=jax.ShapeDtypeStruct((V, B), jnp.float32),
        scratch_shapes=[pltpu.VMEM((B, H), jnp.float32),
                        pltpu.VMEM((H, B), jnp.float32)],
    )(hm, W2, b2.reshape(1, V))
    return outT.T


# speedup vs baseline: 3.0545x; 1.0067x over previous
import functools

import jax
import jax.numpy as jnp
from jax import lax
from jax.experimental import pallas as pl
from jax.experimental.pallas import tpu as pltpu
from jax.experimental.pallas import tpu_sc as plsc


_RB = 2048


def _prepass_body(embt_ref, w1_ref, b1c_ref, out_ref, y_scr):
    y_scr[...] = lax.dot_general(embt_ref[...], w1_ref[...],
                                 (((0,), (0,)), ((), ())),
                                 preferred_element_type=jnp.float32)
    nb = out_ref.shape[0]
    out_ref[...] = jnp.concatenate(
        [y_scr[pl.Slice(k, nb, 8), :] for k in range(8)], axis=1
    ) + b1c_ref[...]


@functools.lru_cache(maxsize=None)
def _make_sc_gather(VG, DG, B):
    D = DG // 8
    info = plsc.get_sparse_core_info()
    NC, NS, L = info.num_cores, info.num_subcores, info.num_lanes
    NW = 16
    b_per_w = B // NW
    n_grp = b_per_w // L
    mesh = plsc.VectorSubcoreMesh(core_axis_name="c", subcore_axis_name="s")

    @functools.partial(
        pl.kernel,
        mesh=mesh,
        out_type=jax.ShapeDtypeStruct((B // 8, DG), jnp.float32),
        scratch_types=[
            pltpu.VMEM((b_per_w,), jnp.int32),
            pltpu.VMEM((b_per_w,), jnp.int32),
            pltpu.VMEM((b_per_w, DG), jnp.float32),
            pltpu.VMEM((b_per_w // 8, DG), jnp.float32),
            pltpu.SemaphoreType.DMA,
        ],
        compiler_params=pltpu.CompilerParams(needs_layout_passes=False),
    )
    def gather_kernel(table_hbm, idx_hbm, out_hbm, idx_v, maj_v, tiles_v,
                      rows_v, sem):
        wid = lax.axis_index("s") * NC + lax.axis_index("c")

        @pl.when(wid < NW)
        def _():
            base = wid * b_per_w
            pltpu.sync_copy(idx_hbm.at[pl.ds(base, b_per_w)], idx_v)
            for g in range(n_grp):
                maj_v[pl.ds(g * L, L)] = jax.lax.shift_right_logical(
                    idx_v[pl.ds(g * L, L)], 3)
            pltpu.async_copy(table_hbm.at[maj_v], tiles_v, sem).wait()
            for g in range(n_grp):
                lane_t = lax.iota(jnp.int32, L) + g * L
                seg = jax.lax.rem(idx_v[pl.ds(g * L, L)], jnp.int32(8)) * D
                dst_r = jax.lax.shift_right_logical(lane_t, 3)
                dst_seg = jax.lax.rem(lane_t, jnp.int32(8)) * D

                def body(c, carry):
                    vals = plsc.load_gather(tiles_v, [lane_t, seg + c])
                    plsc.store_scatter(rows_v, [dst_r, dst_seg + c], vals)
                    return carry

                lax.fori_loop(0, D, body, jnp.int32(0))
            pltpu.sync_copy(
                rows_v, out_hbm.at[pl.ds(wid * (b_per_w // 8), b_per_w // 8)])

    return gather_kernel


_BV = 4096


def _mlp_body(hm_ref, w2_ref, b2_ref, outT_ref, hp_scr, ht_scr):
    @pl.when(pl.program_id(0) == 0)
    def _():
        for k in range(8):
            hp_scr[pl.Slice(k, hm_ref.shape[0], 8), :] = (
                hm_ref[:, pl.ds(k * 64, 64)])
        ht_scr[...] = jnp.maximum(hp_scr[...], 0.0).T

    outT_ref[...] = (
        lax.dot_general(w2_ref[...], ht_scr[...], (((0,), (0,)), ((), ())),
                        preferred_element_type=jnp.float32)
        + b2_ref[...].T
    )


def kernel(x, emb, W1, b1, W2, b2):
    V, H = emb.shape
    B = x.shape[0]
    idx = x.astype(jnp.int32)

    embt = emb.T
    b1cat = jnp.tile(b1, 8).reshape(1, 8 * H)
    table = pl.pallas_call(
        _prepass_body,
        grid=(pl.cdiv(V // 8, _RB),),
        in_specs=[
            pl.BlockSpec((H, 8 * _RB), lambda i: (0, i)),
            pl.BlockSpec((H, H), lambda i: (0, 0)),
            pl.BlockSpec((1, 8 * H), lambda i: (0, 0)),
        ],
        out_specs=pl.BlockSpec((_RB, 8 * H), lambda i: (i, 0)),
        out_shape=jax.ShapeDtypeStruct((V // 8, 8 * H), jnp.float32),
        scratch_shapes=[pltpu.VMEM((8 * _RB, H), jnp.float32)],
    )(embt, W1, b1cat)

    hm = _make_sc_gather(V // 8, 8 * H, B)(table, idx)

    grid = pl.cdiv(V, _BV)
    outT = pl.pallas_call(
        _mlp_body,
        grid=(grid,),
        in_specs=[
            pl.BlockSpec((B // 8, 8 * H), lambda i: (0, 0)),
            pl.BlockSpec((H, _BV), lambda i: (0, i)),
            pl.BlockSpec((1, _BV), lambda i: (0, i)),
        ],
        out_specs=pl.BlockSpec((_BV, B), lambda i: (i, 0)),
        out_shape=jax.ShapeDtypeStruct((V, B), jnp.float32),
        scratch_shapes=[pltpu.VMEM((B, H), jnp.float32),
                        pltpu.VMEM((H, B), jnp.float32)],
    )(hm, W2, b2.reshape(1, V))
    return outT.T


# speedup vs baseline: 3.0997x; 1.0148x over previous
import functools

import jax
import jax.numpy as jnp
from jax import lax
from jax.experimental import pallas as pl
from jax.experimental.pallas import tpu as pltpu
from jax.experimental.pallas import tpu_sc as plsc


_RB = 2048


def _prepass_body(embt_ref, w1_ref, b1c_ref, out_ref, y_scr):
    y_scr[...] = lax.dot_general(embt_ref[...], w1_ref[...],
                                 (((0,), (0,)), ((), ())),
                                 preferred_element_type=jnp.float32)
    nb = out_ref.shape[0]
    out_ref[...] = jnp.concatenate(
        [y_scr[pl.Slice(k, nb, 8), :] for k in range(8)], axis=1
    ) + b1c_ref[...]


@functools.lru_cache(maxsize=None)
def _make_sc_gather(VG, DG, B):
    D = DG // 8
    info = plsc.get_sparse_core_info()
    NC, NS, L = info.num_cores, info.num_subcores, info.num_lanes
    NW = NC * NS
    b_per_w = B // NW
    n_grp = b_per_w // L
    mesh = plsc.VectorSubcoreMesh(core_axis_name="c", subcore_axis_name="s")

    @functools.partial(
        pl.kernel,
        mesh=mesh,
        out_type=jax.ShapeDtypeStruct((B // 8, DG), jnp.float32),
        scratch_types=[
            pltpu.VMEM((b_per_w,), jnp.int32),
            pltpu.VMEM((b_per_w,), jnp.int32),
            pltpu.VMEM((b_per_w, DG), jnp.float32),
            pltpu.VMEM((b_per_w // 8, DG), jnp.float32),
            pltpu.SemaphoreType.DMA,
        ],
        compiler_params=pltpu.CompilerParams(needs_layout_passes=False),
    )
    def gather_kernel(table_hbm, idx_hbm, out_hbm, idx_v, maj_v, tiles_v,
                      rows_v, sem):
        wid = lax.axis_index("s") * NC + lax.axis_index("c")

        @pl.when(wid < NW)
        def _():
            base = wid * b_per_w
            pltpu.sync_copy(idx_hbm.at[pl.ds(base, b_per_w)], idx_v)
            for g in range(n_grp):
                maj_v[pl.ds(g * L, L)] = jax.lax.shift_right_logical(
                    idx_v[pl.ds(g * L, L)], 3)
            pltpu.async_copy(table_hbm.at[maj_v], tiles_v, sem).wait()
            for g in range(n_grp):
                lane_t = lax.iota(jnp.int32, L) + g * L
                seg = jax.lax.rem(idx_v[pl.ds(g * L, L)], jnp.int32(8)) * D
                dst_r = jax.lax.shift_right_logical(lane_t, 3)
                dst_seg = jax.lax.rem(lane_t, jnp.int32(8)) * D

                def body(c, carry):
                    vals = plsc.load_gather(tiles_v, [lane_t, seg + c])
                    plsc.store_scatter(rows_v, [dst_r, dst_seg + c], vals)
                    return carry

                lax.fori_loop(0, D, body, jnp.int32(0))
            pltpu.sync_copy(
                rows_v, out_hbm.at[pl.ds(wid * (b_per_w // 8), b_per_w // 8)])

    return gather_kernel


_BV = 4096


def _mlp_body(hm_ref, w2_ref, b2_ref, outT_ref, hp_scr, ht_scr):
    @pl.when(pl.program_id(0) == 0)
    def _():
        for k in range(8):
            hp_scr[pl.Slice(k, hm_ref.shape[0], 8), :] = (
                hm_ref[:, pl.ds(k * 64, 64)])
        ht_scr[...] = jnp.maximum(hp_scr[...], 0.0).T

    outT_ref[...] = (
        lax.dot_general(w2_ref[...], ht_scr[...], (((0,), (0,)), ((), ())),
                        preferred_element_type=jnp.float32)
        + b2_ref[...].T
    )


def kernel(x, emb, W1, b1, W2, b2):
    V, H = emb.shape
    B = x.shape[0]
    idx = x.astype(jnp.int32)

    embt = emb.T
    b1cat = jnp.tile(b1, 8).reshape(1, 8 * H)
    table = pl.pallas_call(
        _prepass_body,
        grid=(pl.cdiv(V // 8, _RB),),
        in_specs=[
            pl.BlockSpec((H, 8 * _RB), lambda i: (0, i)),
            pl.BlockSpec((H, H), lambda i: (0, 0)),
            pl.BlockSpec((1, 8 * H), lambda i: (0, 0)),
        ],
        out_specs=pl.BlockSpec((_RB, 8 * H), lambda i: (i, 0)),
        out_shape=jax.ShapeDtypeStruct((V // 8, 8 * H), jnp.float32),
        scratch_shapes=[pltpu.VMEM((8 * _RB, H), jnp.float32)],
    )(embt, W1, b1cat)

    hm = _make_sc_gather(V // 8, 8 * H, B)(table, idx)

    grid = pl.cdiv(V, _BV)
    outT = pl.pallas_call(
        _mlp_body,
        grid=(grid,),
        in_specs=[
            pl.BlockSpec((B // 8, 8 * H), lambda i: (0, 0)),
            pl.BlockSpec((H, _BV), lambda i: (0, i)),
            pl.BlockSpec((1, _BV), lambda i: (0, i)),
        ],
        out_specs=pl.BlockSpec((_BV, B), lambda i: (i, 0)),
        out_shape=jax.ShapeDtypeStruct((V, B), jnp.float32),
        scratch_shapes=[pltpu.VMEM((B, H), jnp.float32),
                        pltpu.VMEM((H, B), jnp.float32)],
    )(hm, W2, b2.reshape(1, V))
    return outT.T


# speedup vs baseline: 3.1363x; 1.0118x over previous
import functools

import jax
import jax.numpy as jnp
from jax import lax
from jax.experimental import pallas as pl
from jax.experimental.pallas import tpu as pltpu
from jax.experimental.pallas import tpu_sc as plsc


_RB = 2048


def _prepass_body(embt_ref, w1_ref, b1c_ref, out_ref, y_scr):
    y_scr[...] = lax.dot_general(embt_ref[...].astype(jnp.bfloat16),
                                 w1_ref[...].astype(jnp.bfloat16),
                                 (((0,), (0,)), ((), ())),
                                 preferred_element_type=jnp.float32)
    nb = out_ref.shape[0]
    out_ref[...] = jnp.concatenate(
        [y_scr[pl.Slice(k, nb, 8), :] for k in range(8)], axis=1
    ) + b1c_ref[...]


@functools.lru_cache(maxsize=None)
def _make_sc_gather(VG, DG, B):
    D = DG // 8
    info = plsc.get_sparse_core_info()
    NC, NS, L = info.num_cores, info.num_subcores, info.num_lanes
    NW = NC * NS
    b_per_w = B // NW
    n_grp = b_per_w // L
    mesh = plsc.VectorSubcoreMesh(core_axis_name="c", subcore_axis_name="s")

    @functools.partial(
        pl.kernel,
        mesh=mesh,
        out_type=jax.ShapeDtypeStruct((B // 8, DG), jnp.float32),
        scratch_types=[
            pltpu.VMEM((b_per_w,), jnp.int32),
            pltpu.VMEM((b_per_w,), jnp.int32),
            pltpu.VMEM((b_per_w, DG), jnp.float32),
            pltpu.VMEM((b_per_w // 8, DG), jnp.float32),
            pltpu.SemaphoreType.DMA,
        ],
        compiler_params=pltpu.CompilerParams(needs_layout_passes=False),
    )
    def gather_kernel(table_hbm, idx_hbm, out_hbm, idx_v, maj_v, tiles_v,
                      rows_v, sem):
        wid = lax.axis_index("s") * NC + lax.axis_index("c")

        @pl.when(wid < NW)
        def _():
            base = wid * b_per_w
            pltpu.sync_copy(idx_hbm.at[pl.ds(base, b_per_w)], idx_v)
            for g in range(n_grp):
                maj_v[pl.ds(g * L, L)] = jax.lax.shift_right_logical(
                    idx_v[pl.ds(g * L, L)], 3)
            pltpu.async_copy(table_hbm.at[maj_v], tiles_v, sem).wait()
            for g in range(n_grp):
                lane_t = lax.iota(jnp.int32, L) + g * L
                seg = jax.lax.rem(idx_v[pl.ds(g * L, L)], jnp.int32(8)) * D
                dst_r = jax.lax.shift_right_logical(lane_t, 3)
                dst_seg = jax.lax.rem(lane_t, jnp.int32(8)) * D

                def body(c, carry):
                    vals = plsc.load_gather(tiles_v, [lane_t, seg + c])
                    plsc.store_scatter(rows_v, [dst_r, dst_seg + c], vals)
                    return carry

                lax.fori_loop(0, D, body, jnp.int32(0))
            pltpu.sync_copy(
                rows_v, out_hbm.at[pl.ds(wid * (b_per_w // 8), b_per_w // 8)])

    return gather_kernel


_BV = 4096


def _mlp_body(hm_ref, w2_ref, b2_ref, outT_ref, hp_scr, ht_scr):
    @pl.when(pl.program_id(0) == 0)
    def _():
        for k in range(8):
            hp_scr[pl.Slice(k, hm_ref.shape[0], 8), :] = (
                hm_ref[:, pl.ds(k * 64, 64)])
        ht_scr[...] = jnp.maximum(hp_scr[...], 0.0).T

    outT_ref[...] = (
        lax.dot_general(w2_ref[...], ht_scr[...], (((0,), (0,)), ((), ())),
                        preferred_element_type=jnp.float32)
        + b2_ref[...].T
    )


def kernel(x, emb, W1, b1, W2, b2):
    V, H = emb.shape
    B = x.shape[0]
    idx = x.astype(jnp.int32)

    embt = emb.T
    b1cat = jnp.tile(b1, 8).reshape(1, 8 * H)
    table = pl.pallas_call(
        _prepass_body,
        grid=(pl.cdiv(V // 8, _RB),),
        in_specs=[
            pl.BlockSpec((H, 8 * _RB), lambda i: (0, i)),
            pl.BlockSpec((H, H), lambda i: (0, 0)),
            pl.BlockSpec((1, 8 * H), lambda i: (0, 0)),
        ],
        out_specs=pl.BlockSpec((_RB, 8 * H), lambda i: (i, 0)),
        out_shape=jax.ShapeDtypeStruct((V // 8, 8 * H), jnp.float32),
        scratch_shapes=[pltpu.VMEM((8 * _RB, H), jnp.float32)],
    )(embt, W1, b1cat)

    hm = _make_sc_gather(V // 8, 8 * H, B)(table, idx)

    grid = pl.cdiv(V, _BV)
    outT = pl.pallas_call(
        _mlp_body,
        grid=(grid,),
        in_specs=[
            pl.BlockSpec((B // 8, 8 * H), lambda i: (0, 0)),
            pl.BlockSpec((H, _BV), lambda i: (0, i)),
            pl.BlockSpec((1, _BV), lambda i: (0, i)),
        ],
        out_specs=pl.BlockSpec((_BV, B), lambda i: (i, 0)),
        out_shape=jax.ShapeDtypeStruct((V, B), jnp.float32),
        scratch_shapes=[pltpu.VMEM((B, H), jnp.float32),
                        pltpu.VMEM((H, B), jnp.float32)],
    )(hm, W2, b2.reshape(1, V))
    return outT.T


# speedup vs baseline: 3.1686x; 1.0103x over previous
import functools

import jax
import jax.numpy as jnp
from jax import lax
from jax.experimental import pallas as pl
from jax.experimental.pallas import tpu as pltpu
from jax.experimental.pallas import tpu_sc as plsc


_RB = 2048


def _prepass_body(embt_ref, w1_ref, b1c_ref, out_ref, y_scr):
    y_scr[...] = lax.dot_general(embt_ref[...].astype(jnp.bfloat16),
                                 w1_ref[...].astype(jnp.bfloat16),
                                 (((0,), (0,)), ((), ())),
                                 preferred_element_type=jnp.float32)
    nb = out_ref.shape[0]
    b1c = jnp.concatenate([b1c_ref[...]] * 8, axis=1)
    out_ref[...] = jnp.concatenate(
        [y_scr[pl.Slice(k, nb, 8), :] for k in range(8)], axis=1
    ) + b1c


@functools.lru_cache(maxsize=None)
def _make_sc_gather(VG, DG, B):
    D = DG // 8
    info = plsc.get_sparse_core_info()
    NC, NS, L = info.num_cores, info.num_subcores, info.num_lanes
    NW = NC * NS
    b_per_w = B // NW
    n_grp = b_per_w // L
    mesh = plsc.VectorSubcoreMesh(core_axis_name="c", subcore_axis_name="s")

    @functools.partial(
        pl.kernel,
        mesh=mesh,
        out_type=jax.ShapeDtypeStruct((B // 8, DG), jnp.float32),
        scratch_types=[
            pltpu.VMEM((b_per_w,), jnp.int32),
            pltpu.VMEM((b_per_w,), jnp.int32),
            pltpu.VMEM((b_per_w, DG), jnp.float32),
            pltpu.VMEM((b_per_w // 8, DG), jnp.float32),
            pltpu.SemaphoreType.DMA,
        ],
        compiler_params=pltpu.CompilerParams(needs_layout_passes=False),
    )
    def gather_kernel(table_hbm, idx_hbm, out_hbm, idx_v, maj_v, tiles_v,
                      rows_v, sem):
        wid = lax.axis_index("s") * NC + lax.axis_index("c")

        @pl.when(wid < NW)
        def _():
            base = wid * b_per_w
            pltpu.sync_copy(idx_hbm.at[pl.ds(base, b_per_w)], idx_v)
            for g in range(n_grp):
                maj_v[pl.ds(g * L, L)] = jax.lax.shift_right_logical(
                    idx_v[pl.ds(g * L, L)], 3)
            pltpu.async_copy(table_hbm.at[maj_v], tiles_v, sem).wait()
            for g in range(n_grp):
                lane_t = lax.iota(jnp.int32, L) + g * L
                seg = jax.lax.rem(idx_v[pl.ds(g * L, L)], jnp.int32(8)) * D
                dst_r = jax.lax.shift_right_logical(lane_t, 3)
                dst_seg = jax.lax.rem(lane_t, jnp.int32(8)) * D

                def body(c, carry):
                    vals = plsc.load_gather(tiles_v, [lane_t, seg + c])
                    plsc.store_scatter(rows_v, [dst_r, dst_seg + c], vals)
                    return carry

                lax.fori_loop(0, D, body, jnp.int32(0))
            pltpu.sync_copy(
                rows_v, out_hbm.at[pl.ds(wid * (b_per_w // 8), b_per_w // 8)])

    return gather_kernel


_BV = 4096


def _mlp_body(hm_ref, w2_ref, b2_ref, outT_ref, hp_scr, ht_scr):
    @pl.when(pl.program_id(0) == 0)
    def _():
        for k in range(8):
            hp_scr[pl.Slice(k, hm_ref.shape[0], 8), :] = (
                hm_ref[:, pl.ds(k * 64, 64)])
        ht_scr[...] = jnp.maximum(hp_scr[...], 0.0).T

    outT_ref[...] = (
        lax.dot_general(w2_ref[...], ht_scr[...], (((0,), (0,)), ((), ())),
                        preferred_element_type=jnp.float32)
        + b2_ref[...].T
    )


def kernel(x, emb, W1, b1, W2, b2):
    V, H = emb.shape
    B = x.shape[0]
    idx = x.astype(jnp.int32)

    embt = emb.T
    b1cat = b1.reshape(1, H)
    table = pl.pallas_call(
        _prepass_body,
        grid=(pl.cdiv(V // 8, _RB),),
        in_specs=[
            pl.BlockSpec((H, 8 * _RB), lambda i: (0, i)),
            pl.BlockSpec((H, H), lambda i: (0, 0)),
            pl.BlockSpec((1, H), lambda i: (0, 0)),
        ],
        out_specs=pl.BlockSpec((_RB, 8 * H), lambda i: (i, 0)),
        out_shape=jax.ShapeDtypeStruct((V // 8, 8 * H), jnp.float32),
        scratch_shapes=[pltpu.VMEM((8 * _RB, H), jnp.float32)],
    )(embt, W1, b1cat)

    hm = _make_sc_gather(V // 8, 8 * H, B)(table, idx)

    grid = pl.cdiv(V, _BV)
    outT = pl.pallas_call(
        _mlp_body,
        grid=(grid,),
        in_specs=[
            pl.BlockSpec((B // 8, 8 * H), lambda i: (0, 0)),
            pl.BlockSpec((H, _BV), lambda i: (0, i)),
            pl.BlockSpec((1, _BV), lambda i: (0, i)),
        ],
        out_specs=pl.BlockSpec((_BV, B), lambda i: (i, 0)),
        out_shape=jax.ShapeDtypeStruct((V, B), jnp.float32),
        scratch_shapes=[pltpu.VMEM((B, H), jnp.float32),
                        pltpu.VMEM((H, B), jnp.float32)],
    )(hm, W2, b2.reshape(1, V))
    return outT.T
